# baseline (speedup 1.0000x reference)
import functools

import jax
import jax.numpy as jnp
from jax.experimental import pallas as pl
from jax.experimental.pallas import tpu as pltpu

B, S, D, DFF, E, TOP_K = 64, 1, 768, 3072, 16, 2
N = B * S
DH = D // 2
FH = DFF // 2
FQ = DFF // 4


def _moe_kernel(x_ref, wg_ref, bg_ref, w1a_ref, w1b_ref, b1_ref,
                w2a_ref, w2b_ref, b2_ref, alpha_ref, out_ref, w_scr, h_scr):
    e = pl.program_id(0)
    c = pl.program_id(1)

    @pl.when((e == 0) & (c == 0))
    def _gate():
        x = x_ref[...]
        logits = jnp.dot(x, wg_ref[...], preferred_element_type=jnp.float32)
        logits = logits + bg_ref[0, :][None, :]
        m = jnp.max(logits, axis=-1, keepdims=True)
        ex = jnp.exp(logits - m)
        probs = ex / jnp.sum(ex, axis=-1, keepdims=True)
        ids = jax.lax.broadcasted_iota(jnp.int32, (N, E), 1)
        s1 = jnp.max(probs, axis=-1, keepdims=True)
        i1 = jnp.min(jnp.where(probs == s1, ids, E), axis=-1, keepdims=True)
        probs2 = jnp.where(ids == i1, -jnp.inf, probs)
        s2 = jnp.max(probs2, axis=-1, keepdims=True)
        i2 = jnp.min(jnp.where(probs2 == s2, ids, E), axis=-1, keepdims=True)
        w = jnp.where(ids == i1, s1, 0.0) + jnp.where(ids == i2, s2, 0.0)
        w_scr[...] = w * alpha_ref[0, :][None, :]

    def gelu(v):
        return 0.5 * v * (1.0 + jax.lax.erf(v * 0.7071067811865476))

    @pl.when(c == 0)
    def _p0():
        xl = x_ref[:, :DH]
        h = jnp.concatenate(
            [jnp.dot(xl, w1a_ref[0], preferred_element_type=jnp.float32),
             jnp.dot(xl, w1b_ref[0], preferred_element_type=jnp.float32)],
            axis=1)
        h_scr[...] = h

    @pl.when(c == 1)
    def _p1():
        xh = x_ref[:, DH:]
        h = jnp.concatenate(
            [jnp.dot(xh, w1a_ref[0], preferred_element_type=jnp.float32),
             jnp.dot(xh, w1b_ref[0], preferred_element_type=jnp.float32)],
            axis=1)
        h_scr[...] = h_scr[...] + h + b1_ref[0]

    @pl.when(c >= 2)
    def _p23():
        ids = jax.lax.broadcasted_iota(jnp.int32, (N, E), 1)
        wcol = jnp.sum(jnp.where(ids == e, w_scr[...], 0.0), axis=-1,
                       keepdims=True)
        base = jnp.where(c == 2, 0, FH)
        g = gelu(h_scr[:, pl.ds(base, FH)])
        o = jnp.dot(g[:, :FQ], w2a_ref[0], preferred_element_type=jnp.float32)
        o = o + jnp.dot(g[:, FQ:], w2b_ref[0],
                        preferred_element_type=jnp.float32)
        o = jnp.where(c == 2, o + b2_ref[0], o)
        o = o * wcol

        @pl.when((e == 0) & (c == 2))
        def _init():
            out_ref[...] = o

        @pl.when((e > 0) | (c > 2))
        def _acc():
            out_ref[...] += o


@functools.partial(jax.jit, static_argnames=("interpret",))
def _moe(x, Wg, bg2, W1, b1, W2, b2, alpha2, interpret=False):
    def w1a_map(e, c):
        adv = c >= 2
        en = jnp.where(adv & (e < E - 1), e + 1, e)
        ch = jnp.where(adv, jnp.where(e < E - 1, 0, 1), c)
        return (en, ch, 0)

    def w1b_map(e, c):
        adv = c >= 2
        en = jnp.where(adv & (e < E - 1), e + 1, e)
        ch = jnp.where(adv, jnp.where(e < E - 1, 0, 1), c)
        return (en, ch, 1)

    def w2a_map(e, c):
        return (e, jnp.where(c == 3, 2, 0), 0)

    def w2b_map(e, c):
        return (e, jnp.where(c == 3, 3, 1), 0)

    return pl.pallas_call(
        _moe_kernel,
        grid=(E, 4),
        in_specs=[
            pl.BlockSpec((N, D), lambda e, c: (0, 0)),
            pl.BlockSpec((D, E), lambda e, c: (0, 0)),
            pl.BlockSpec((1, E), lambda e, c: (0, 0)),
            pl.BlockSpec((1, DH, FH), w1a_map),
            pl.BlockSpec((1, DH, FH), w1b_map),
            pl.BlockSpec((1, 1, DFF), lambda e, c: (e, 0, 0)),
            pl.BlockSpec((1, FQ, D), w2a_map),
            pl.BlockSpec((1, FQ, D), w2b_map),
            pl.BlockSpec((1, 1, D), lambda e, c: (e, 0, 0)),
            pl.BlockSpec((1, E), lambda e, c: (0, 0)),
        ],
        out_specs=pl.BlockSpec((N, D), lambda e, c: (0, 0)),
        out_shape=jax.ShapeDtypeStruct((N, D), jnp.float32),
        scratch_shapes=[pltpu.VMEM((N, E), jnp.float32),
                        pltpu.VMEM((N, DFF), jnp.float32)],
        compiler_params=pltpu.CompilerParams(
            dimension_semantics=("arbitrary", "arbitrary"),
        ),
        interpret=interpret,
    )(x, Wg, bg2, W1, W1, b1, W2, W2, b2, alpha2)


def kernel(hidden_states, Wg, bg, W1, b1, W2, b2, alpha):
    b, s, d = hidden_states.shape
    x = hidden_states.reshape(-1, d)
    out = _moe(x, Wg, bg.reshape(1, E), W1, b1.reshape(E, 1, DFF), W2,
               b2.reshape(E, 1, D), alpha.reshape(1, E))
    return out.reshape(b, s, d)


# speedup vs baseline: 1.0933x; 1.0933x over previous
import functools

import jax
import jax.numpy as jnp
from jax.experimental import pallas as pl
from jax.experimental.pallas import tpu as pltpu

B, S, D, DFF, E, TOP_K = 64, 1, 768, 3072, 16, 2
N = B * S
H = DFF // 2


def _moe_kernel(x_ref, wg_ref, bg_ref, w1a_ref, w1b_ref, b1_ref,
                w2a_ref, w2b_ref, b2_ref, alpha_ref, out_ref, w_scr):
    e = pl.program_id(0)

    @pl.when(e == 0)
    def _gate():
        x = x_ref[...]
        logits = jnp.dot(x, wg_ref[...], preferred_element_type=jnp.float32)
        logits = logits + bg_ref[0, :][None, :]
        m = jnp.max(logits, axis=-1, keepdims=True)
        ex = jnp.exp(logits - m)
        probs = ex / jnp.sum(ex, axis=-1, keepdims=True)
        ids = jax.lax.broadcasted_iota(jnp.int32, (N, E), 1)
        s1 = jnp.max(probs, axis=-1, keepdims=True)
        i1 = jnp.min(jnp.where(probs == s1, ids, E), axis=-1, keepdims=True)
        probs2 = jnp.where(ids == i1, -jnp.inf, probs)
        s2 = jnp.max(probs2, axis=-1, keepdims=True)
        i2 = jnp.min(jnp.where(probs2 == s2, ids, E), axis=-1, keepdims=True)
        w = jnp.where(ids == i1, s1, 0.0) + jnp.where(ids == i2, s2, 0.0)
        w_scr[...] = w * alpha_ref[0, :][None, :]

    def gelu(v):
        return 0.5 * v * (1.0 + jax.lax.erf(v * 0.7071067811865476))

    x = x_ref[...]
    ha = jnp.dot(x, w1a_ref[0], preferred_element_type=jnp.float32)
    hb = jnp.dot(x, w1b_ref[0], preferred_element_type=jnp.float32)
    ga = gelu(ha + b1_ref[0, :, :H])
    gb = gelu(hb + b1_ref[0, :, H:])
    o = jnp.dot(ga, w2a_ref[0], preferred_element_type=jnp.float32)
    o = o + jnp.dot(gb, w2b_ref[0], preferred_element_type=jnp.float32)
    o = o + b2_ref[0]
    ids = jax.lax.broadcasted_iota(jnp.int32, (N, E), 1)
    wcol = jnp.sum(jnp.where(ids == e, w_scr[...], 0.0), axis=-1,
                   keepdims=True)
    o = o * wcol

    @pl.when(e == 0)
    def _init():
        out_ref[...] = o

    @pl.when(e > 0)
    def _acc():
        out_ref[...] += o


@functools.partial(jax.jit, static_argnames=("interpret",))
def _moe(x, Wg, bg2, W1, b1, W2, b2, alpha2, interpret=False):
    return pl.pallas_call(
        _moe_kernel,
        grid=(E,),
        in_specs=[
            pl.BlockSpec((N, D), lambda e: (0, 0)),
            pl.BlockSpec((D, E), lambda e: (0, 0)),
            pl.BlockSpec((1, E), lambda e: (0, 0)),
            pl.BlockSpec((1, D, H), lambda e: (e, 0, 0)),
            pl.BlockSpec((1, D, H), lambda e: (e, 0, 1)),
            pl.BlockSpec((1, 1, DFF), lambda e: (e, 0, 0)),
            pl.BlockSpec((1, H, D), lambda e: (e, 0, 0)),
            pl.BlockSpec((1, H, D), lambda e: (e, 1, 0)),
            pl.BlockSpec((1, 1, D), lambda e: (e, 0, 0)),
            pl.BlockSpec((1, E), lambda e: (0, 0)),
        ],
        out_specs=pl.BlockSpec((N, D), lambda e: (0, 0)),
        out_shape=jax.ShapeDtypeStruct((N, D), jnp.float32),
        scratch_shapes=[pltpu.VMEM((N, E), jnp.float32)],
        compiler_params=pltpu.CompilerParams(
            dimension_semantics=("arbitrary",),
        ),
        interpret=interpret,
    )(x, Wg, bg2, W1, W1, b1, W2, W2, b2, alpha2)


def kernel(hidden_states, Wg, bg, W1, b1, W2, b2, alpha):
    b, s, d = hidden_states.shape
    x = hidden_states.reshape(-1, d)
    out = _moe(x, Wg, bg.reshape(1, E), W1, b1.reshape(E, 1, DFF), W2,
               b2.reshape(E, 1, D), alpha.reshape(1, E))
    return out.reshape(b, s, d)


# speedup vs baseline: 1.0942x; 1.0008x over previous
import functools

import jax
import jax.numpy as jnp
from jax.experimental import pallas as pl
from jax.experimental.pallas import tpu as pltpu

B, S, D, DFF, E, TOP_K = 64, 1, 768, 3072, 16, 2
N = B * S
DH = D // 2
FH = DFF // 2
NB = 4
NCHUNK = 2 * E


def _moe_kernel(x_ref, wg_ref, bg_ref, w1_hbm, b1_ref, w2_hbm, b2_ref,
                alpha_ref, out_ref, w1_buf, w2_buf, w1_sem, w2_sem):
    def w1_copy(k):
        e, c = divmod(k, 2)
        return pltpu.make_async_copy(
            w1_hbm.at[e, pl.ds(c * DH, DH), :], w1_buf.at[k % NB],
            w1_sem.at[k % NB])

    def w2_copy(k):
        e, c = divmod(k, 2)
        return pltpu.make_async_copy(
            w2_hbm.at[e, pl.ds(c * FH, FH), :], w2_buf.at[k % NB],
            w2_sem.at[k % NB])

    for k in range(NB):
        w1_copy(k).start()
        w2_copy(k).start()

    x = x_ref[...]
    logits = jnp.dot(x, wg_ref[...], preferred_element_type=jnp.float32)
    logits = logits + bg_ref[0, :][None, :]
    m = jnp.max(logits, axis=-1, keepdims=True)
    ex = jnp.exp(logits - m)
    probs = ex / jnp.sum(ex, axis=-1, keepdims=True)
    ids = jax.lax.broadcasted_iota(jnp.int32, (N, E), 1)
    s1 = jnp.max(probs, axis=-1, keepdims=True)
    i1 = jnp.min(jnp.where(probs == s1, ids, E), axis=-1, keepdims=True)
    probs2 = jnp.where(ids == i1, -jnp.inf, probs)
    s2 = jnp.max(probs2, axis=-1, keepdims=True)
    i2 = jnp.min(jnp.where(probs2 == s2, ids, E), axis=-1, keepdims=True)
    w = jnp.where(ids == i1, s1, 0.0) + jnp.where(ids == i2, s2, 0.0)
    w = w * alpha_ref[0, :][None, :]

    def gelu(v):
        return 0.5 * v * (1.0 + jax.lax.erf(v * 0.7071067811865476))

    xl, xh = x[:, :DH], x[:, DH:]
    for e in range(E):
        k0, k1 = 2 * e, 2 * e + 1
        w1_copy(k0).wait()
        w1_copy(k1).wait()
        h = jnp.dot(xl, w1_buf[k0 % NB], preferred_element_type=jnp.float32)
        h = h + jnp.dot(xh, w1_buf[k1 % NB],
                        preferred_element_type=jnp.float32)
        h = h + b1_ref[e][None, :]
        if k0 + NB < NCHUNK:
            w1_copy(k0 + NB).start()
        if k1 + NB < NCHUNK:
            w1_copy(k1 + NB).start()
        g = gelu(h)
        w2_copy(k0).wait()
        w2_copy(k1).wait()
        o = jnp.dot(g[:, :FH], w2_buf[k0 % NB],
                    preferred_element_type=jnp.float32)
        o = o + jnp.dot(g[:, FH:], w2_buf[k1 % NB],
                        preferred_element_type=jnp.float32)
        o = (o + b2_ref[e][None, :]) * w[:, e:e + 1]
        if k0 + NB < NCHUNK:
            w2_copy(k0 + NB).start()
        if k1 + NB < NCHUNK:
            w2_copy(k1 + NB).start()
        if e == 0:
            out_ref[...] = o
        else:
            out_ref[...] += o


@functools.partial(jax.jit, static_argnames=("interpret",))
def _moe(x, Wg, bg2, W1, b1, W2, b2, alpha2, interpret=False):
    return pl.pallas_call(
        _moe_kernel,
        in_specs=[
            pl.BlockSpec(memory_space=pltpu.VMEM),
            pl.BlockSpec(memory_space=pltpu.VMEM),
            pl.BlockSpec(memory_space=pltpu.VMEM),
            pl.BlockSpec(memory_space=pltpu.HBM),
            pl.BlockSpec(memory_space=pltpu.VMEM),
            pl.BlockSpec(memory_space=pltpu.HBM),
            pl.BlockSpec(memory_space=pltpu.VMEM),
            pl.BlockSpec(memory_space=pltpu.VMEM),
        ],
        out_specs=pl.BlockSpec(memory_space=pltpu.VMEM),
        out_shape=jax.ShapeDtypeStruct((N, D), jnp.float32),
        scratch_shapes=[
            pltpu.VMEM((NB, DH, DFF), jnp.float32),
            pltpu.VMEM((NB, FH, D), jnp.float32),
            pltpu.SemaphoreType.DMA((NB,)),
            pltpu.SemaphoreType.DMA((NB,)),
        ],
        interpret=interpret,
    )(x, Wg, bg2, W1, b1, W2, b2, alpha2)


def kernel(hidden_states, Wg, bg, W1, b1, W2, b2, alpha):
    b, s, d = hidden_states.shape
    x = hidden_states.reshape(-1, d)
    out = _moe(x, Wg, bg.reshape(1, E), W1, b1, W2, b2,
               alpha.reshape(1, E))
    return out.reshape(b, s, d)


# speedup vs baseline: 1.1178x; 1.0216x over previous
import functools

import jax
import jax.numpy as jnp
from jax.experimental import pallas as pl
from jax.experimental.pallas import tpu as pltpu

B, S, D, DFF, E, TOP_K = 64, 1, 768, 3072, 16, 2
N = B * S
DH = D // 2
FH = DFF // 2
NB = 4
NCHUNK = 2 * E


def _moe_kernel(x_ref, wg_ref, bg_ref, w1_hbm, b1_ref, w2_hbm, b2_ref,
                alpha_ref, out_ref, w1_buf, w2_buf, w1_sem, w2_sem):
    def w1_copy(k):
        e, c = divmod(k, 2)
        return pltpu.make_async_copy(
            w1_hbm.at[e, pl.ds(c * DH, DH), :], w1_buf.at[k % NB],
            w1_sem.at[k % NB])

    def w2_copy(k):
        e, c = divmod(k, 2)
        return pltpu.make_async_copy(
            w2_hbm.at[e, pl.ds(c * FH, FH), :], w2_buf.at[k % NB],
            w2_sem.at[k % NB])

    for k in range(NB):
        w1_copy(k).start()
        w2_copy(k).start()

    x = x_ref[...]
    logits = jnp.dot(x, wg_ref[...], preferred_element_type=jnp.float32)
    logits = logits + bg_ref[0, :][None, :]
    m = jnp.max(logits, axis=-1, keepdims=True)
    ex = jnp.exp(logits - m)
    probs = ex / jnp.sum(ex, axis=-1, keepdims=True)
    ids = jax.lax.broadcasted_iota(jnp.int32, (N, E), 1)
    s1 = jnp.max(probs, axis=-1, keepdims=True)
    i1 = jnp.min(jnp.where(probs == s1, ids, E), axis=-1, keepdims=True)
    probs2 = jnp.where(ids == i1, -jnp.inf, probs)
    s2 = jnp.max(probs2, axis=-1, keepdims=True)
    i2 = jnp.min(jnp.where(probs2 == s2, ids, E), axis=-1, keepdims=True)
    w = jnp.where(ids == i1, s1, 0.0) + jnp.where(ids == i2, s2, 0.0)
    w = w * alpha_ref[0, :][None, :]

    def gelu(v):
        return 0.5 * v * (1.0 + jax.lax.erf(v * 0.7071067811865476))

    xl = x[:, :DH].astype(jnp.bfloat16)
    xh = x[:, DH:].astype(jnp.bfloat16)
    for e in range(E):
        k0, k1 = 2 * e, 2 * e + 1
        w1_copy(k0).wait()
        w1_copy(k1).wait()
        h = jnp.dot(xl, w1_buf[k0 % NB].astype(jnp.bfloat16),
                    preferred_element_type=jnp.float32)
        h = h + jnp.dot(xh, w1_buf[k1 % NB].astype(jnp.bfloat16),
                        preferred_element_type=jnp.float32)
        h = h + b1_ref[e][None, :]
        if k0 + NB < NCHUNK:
            w1_copy(k0 + NB).start()
        if k1 + NB < NCHUNK:
            w1_copy(k1 + NB).start()
        g = gelu(h)
        w2_copy(k0).wait()
        w2_copy(k1).wait()
        o = jnp.dot(g[:, :FH].astype(jnp.bfloat16),
                    w2_buf[k0 % NB].astype(jnp.bfloat16),
                    preferred_element_type=jnp.float32)
        o = o + jnp.dot(g[:, FH:].astype(jnp.bfloat16),
                        w2_buf[k1 % NB].astype(jnp.bfloat16),
                        preferred_element_type=jnp.float32)
        o = (o + b2_ref[e][None, :]) * w[:, e:e + 1]
        if k0 + NB < NCHUNK:
            w2_copy(k0 + NB).start()
        if k1 + NB < NCHUNK:
            w2_copy(k1 + NB).start()
        if e == 0:
            out_ref[...] = o
        else:
            out_ref[...] += o


@functools.partial(jax.jit, static_argnames=("interpret",))
def _moe(x, Wg, bg2, W1, b1, W2, b2, alpha2, interpret=False):
    return pl.pallas_call(
        _moe_kernel,
        in_specs=[
            pl.BlockSpec(memory_space=pltpu.VMEM),
            pl.BlockSpec(memory_space=pltpu.VMEM),
            pl.BlockSpec(memory_space=pltpu.VMEM),
            pl.BlockSpec(memory_space=pltpu.HBM),
            pl.BlockSpec(memory_space=pltpu.VMEM),
            pl.BlockSpec(memory_space=pltpu.HBM),
            pl.BlockSpec(memory_space=pltpu.VMEM),
            pl.BlockSpec(memory_space=pltpu.VMEM),
        ],
        out_specs=pl.BlockSpec(memory_space=pltpu.VMEM),
        out_shape=jax.ShapeDtypeStruct((N, D), jnp.float32),
        scratch_shapes=[
            pltpu.VMEM((NB, DH, DFF), jnp.float32),
            pltpu.VMEM((NB, FH, D), jnp.float32),
            pltpu.SemaphoreType.DMA((NB,)),
            pltpu.SemaphoreType.DMA((NB,)),
        ],
        interpret=interpret,
    )(x, Wg, bg2, W1, b1, W2, b2, alpha2)


def kernel(hidden_states, Wg, bg, W1, b1, W2, b2, alpha):
    b, s, d = hidden_states.shape
    x = hidden_states.reshape(-1, d)
    out = _moe(x, Wg, bg.reshape(1, E), W1, b1, W2, b2,
               alpha.reshape(1, E))
    return out.reshape(b, s, d)
